# baseline (speedup 1.0000x reference)
import functools

import jax
import jax.numpy as jnp
from jax import lax
from jax.experimental import pallas as pl
from jax.experimental.pallas import tpu as pltpu
from jax.experimental.pallas import tpu_sc as plsc

K = 64
N = 1000000
B = 16384
NC = 2
NS = 16
NW = NC * NS
BPW = B // NW
CH = 128
L = 16
NCH = BPW // CH
NG = CH // L
SCALE = 1.0 / (K ** 0.5)

TBLK = 1024
TGRID = (N + TBLK - 1) // TBLK


def _transpose_body(in_ref, out_ref):
    eye_pad = (lax.broadcasted_iota(jnp.int32, (K, 2 * K), 0)
               == lax.broadcasted_iota(jnp.int32, (K, 2 * K), 1)
               ).astype(jnp.float32)
    out_ref[...] = lax.dot_general(
        in_ref[...], eye_pad, (((0,), (0,)), ((), ())),
        preferred_element_type=jnp.float32)


_stage = pl.pallas_call(
    _transpose_body,
    grid=(TGRID,),
    in_specs=[pl.BlockSpec((K, TBLK), lambda i: (0, i))],
    out_specs=pl.BlockSpec((TBLK, 2 * K), lambda i: (i, 0)),
    out_shape=jax.ShapeDtypeStruct((N, 2 * K), jnp.float32),
)


def _pair_dots(idx_a_hbm, idx_b_hbm, tab_a, tab_b, base, ia_v, ib_v, a_v, b_v,
               out_v, sem, combine):
    pltpu.sync_copy(idx_a_hbm.at[pl.ds(base, BPW)], ia_v)
    pltpu.sync_copy(idx_b_hbm.at[pl.ds(base, BPW)], ib_v)

    lanes = lax.iota(jnp.int32, L)

    def chunk_body(c, _):
        off = c * CH
        cp_a = pltpu.async_copy(tab_a.at[ia_v.at[pl.ds(off, CH)]], a_v, sem)
        cp_b = pltpu.async_copy(tab_b.at[ib_v.at[pl.ds(off, CH)]], b_v, sem)
        cp_a.wait()
        cp_b.wait()

        def grp_body(g, _):
            r0 = g * L
            vec = jnp.zeros((L,), jnp.float32)
            for r in range(L):
                acc = a_v[r0 + r, pl.ds(0, L)] * b_v[r0 + r, pl.ds(0, L)]
                for j in range(1, K // L):
                    acc = acc + (a_v[r0 + r, pl.ds(j * L, L)]
                                 * b_v[r0 + r, pl.ds(j * L, L)])
                for step in (8, 4, 2, 1):
                    acc = acc + acc.at[lanes ^ step].get(
                        mode="promise_in_bounds")
                vec = jnp.where(lanes == r, acc, vec)
            combine(pl.ds(off + r0, L), vec)
            return _

        return lax.fori_loop(0, NG, grp_body, None)

    lax.fori_loop(0, NCH, chunk_body, None)


def _mf_body(uid_hbm, iid_hbm, ui_hbm, iu_hbm, out_hbm,
             ia_v, ib_v, a_v, b_v, out_v, sem):
    wid = lax.axis_index("s") * NC + lax.axis_index("c")
    base = wid * BPW

    def combine(dst, vec):
        out_v[dst] = vec

    _pair_dots(uid_hbm, iid_hbm, ui_hbm, iu_hbm, base, ia_v, ib_v, a_v, b_v,
               out_v, sem, combine)
    pltpu.sync_copy(out_v, out_hbm.at[pl.ds(base, BPW)])


def _fmc_body(iid_hbm, bp_hbm, il_hbm, li_hbm, mf_hbm, out_hbm,
              ia_v, ib_v, a_v, b_v, out_v, sem):
    wid = lax.axis_index("s") * NC + lax.axis_index("c")
    base = wid * BPW
    pltpu.sync_copy(mf_hbm.at[pl.ds(base, BPW)], out_v)

    def combine(dst, vec):
        out_v[dst] = (out_v[dst] + vec) * SCALE

    _pair_dots(iid_hbm, bp_hbm, il_hbm, li_hbm, base, ia_v, ib_v, a_v, b_v,
               out_v, sem, combine)
    pltpu.sync_copy(out_v, out_hbm.at[pl.ds(base, BPW)])


_SCRATCH = [
    pltpu.VMEM((BPW,), jnp.int32),
    pltpu.VMEM((BPW,), jnp.int32),
    pltpu.VMEM((CH, 2 * K), jnp.float32),
    pltpu.VMEM((CH, 2 * K), jnp.float32),
    pltpu.VMEM((BPW,), jnp.float32),
    pltpu.SemaphoreType.DMA,
]

_mf = functools.partial(
    pl.kernel,
    mesh=plsc.VectorSubcoreMesh(core_axis_name="c", subcore_axis_name="s"),
    compiler_params=pltpu.CompilerParams(use_tc_tiling_on_sc=True),
    out_type=jax.ShapeDtypeStruct((B,), jnp.float32),
    scratch_types=_SCRATCH,
)(_mf_body)

_fmc = functools.partial(
    pl.kernel,
    mesh=plsc.VectorSubcoreMesh(core_axis_name="c", subcore_axis_name="s"),
    compiler_params=pltpu.CompilerParams(use_tc_tiling_on_sc=True),
    out_type=jax.ShapeDtypeStruct((B,), jnp.float32),
    scratch_types=_SCRATCH,
)(_fmc_body)


def kernel(uid, basket_prev, iid, UI, IU, IL, LI):
    uid = uid.astype(jnp.int32)
    bp = basket_prev.astype(jnp.int32)
    iid = iid.astype(jnp.int32)
    ui_s = _stage(UI.T)
    iu_s = _stage(IU.T)
    il_s = _stage(IL.T)
    li_s = _stage(LI.T)
    mf = _mf(uid, iid, ui_s, iu_s)
    return _fmc(iid, bp, il_s, li_s, mf)


# speedup vs baseline: 2.3064x; 2.3064x over previous
import functools

import jax
import jax.numpy as jnp
from jax import lax
from jax.experimental import pallas as pl
from jax.experimental.pallas import tpu as pltpu
from jax.experimental.pallas import tpu_sc as plsc

K = 64
N = 1000000
B = 16384
NC = 2
NS = 16
NW = NC * NS
BPW = B // NW
CH = 128
L = 16
NCH = BPW // CH
NG = CH // L
SCALE = 1.0 / (K ** 0.5)

TBLK = 2048
TGRID = (N + TBLK - 1) // TBLK


def _stage_body(a_ref, b_ref, out_ref):
    out_ref[:, 0:K] = jnp.transpose(a_ref[...], (1, 0))
    out_ref[:, K:2 * K] = jnp.transpose(b_ref[...], (1, 0))


_stage = pl.pallas_call(
    _stage_body,
    grid=(TGRID,),
    in_specs=[pl.BlockSpec((K, TBLK), lambda i: (0, i)),
              pl.BlockSpec((K, TBLK), lambda i: (0, i))],
    out_specs=pl.BlockSpec((TBLK, 2 * K), lambda i: (i, 0)),
    out_shape=jax.ShapeDtypeStruct((N, 2 * K), jnp.float32),
)


def _pair_dots(idx_a_hbm, idx_b_hbm, tab, base, ia_v, ib_v, a_v, b_v,
               out_v, sem, combine):
    pltpu.sync_copy(idx_a_hbm.at[pl.ds(base, BPW)], ia_v)
    pltpu.sync_copy(idx_b_hbm.at[pl.ds(base, BPW)], ib_v)

    lanes = lax.iota(jnp.int32, L)

    def chunk_body(c, _):
        off = c * CH
        cp_a = pltpu.async_copy(tab.at[ia_v.at[pl.ds(off, CH)]], a_v, sem)
        cp_b = pltpu.async_copy(tab.at[ib_v.at[pl.ds(off, CH)]], b_v, sem)
        cp_a.wait()
        cp_b.wait()

        def grp_body(g, _):
            r0 = g * L
            vec = jnp.zeros((L,), jnp.float32)
            for r in range(L):
                acc = a_v[r0 + r, pl.ds(0, L)] * b_v[r0 + r, pl.ds(K, L)]
                for j in range(1, K // L):
                    acc = acc + (a_v[r0 + r, pl.ds(j * L, L)]
                                 * b_v[r0 + r, pl.ds(K + j * L, L)])
                for step in (8, 4, 2, 1):
                    acc = acc + acc.at[lanes ^ step].get(
                        mode="promise_in_bounds")
                vec = jnp.where(lanes == r, acc, vec)
            combine(pl.ds(off + r0, L), vec)
            return _

        return lax.fori_loop(0, NG, grp_body, None)

    lax.fori_loop(0, NCH, chunk_body, None)


def _mf_body(uid_hbm, iid_hbm, tab_hbm, out_hbm,
             ia_v, ib_v, a_v, b_v, out_v, sem):
    wid = lax.axis_index("s") * NC + lax.axis_index("c")
    base = wid * BPW

    def combine(dst, vec):
        out_v[dst] = vec

    _pair_dots(uid_hbm, iid_hbm, tab_hbm, base, ia_v, ib_v, a_v, b_v,
               out_v, sem, combine)
    pltpu.sync_copy(out_v, out_hbm.at[pl.ds(base, BPW)])


def _fmc_body(iid_hbm, bp_hbm, tab_hbm, mf_hbm, out_hbm,
              ia_v, ib_v, a_v, b_v, out_v, sem):
    wid = lax.axis_index("s") * NC + lax.axis_index("c")
    base = wid * BPW
    pltpu.sync_copy(mf_hbm.at[pl.ds(base, BPW)], out_v)

    def combine(dst, vec):
        out_v[dst] = (out_v[dst] + vec) * SCALE

    _pair_dots(iid_hbm, bp_hbm, tab_hbm, base, ia_v, ib_v, a_v, b_v,
               out_v, sem, combine)
    pltpu.sync_copy(out_v, out_hbm.at[pl.ds(base, BPW)])


_SCRATCH = [
    pltpu.VMEM((BPW,), jnp.int32),
    pltpu.VMEM((BPW,), jnp.int32),
    pltpu.VMEM((CH, 2 * K), jnp.float32),
    pltpu.VMEM((CH, 2 * K), jnp.float32),
    pltpu.VMEM((BPW,), jnp.float32),
    pltpu.SemaphoreType.DMA,
]

_mf = functools.partial(
    pl.kernel,
    mesh=plsc.VectorSubcoreMesh(core_axis_name="c", subcore_axis_name="s"),
    compiler_params=pltpu.CompilerParams(use_tc_tiling_on_sc=True),
    out_type=jax.ShapeDtypeStruct((B,), jnp.float32),
    scratch_types=_SCRATCH,
)(_mf_body)

_fmc = functools.partial(
    pl.kernel,
    mesh=plsc.VectorSubcoreMesh(core_axis_name="c", subcore_axis_name="s"),
    compiler_params=pltpu.CompilerParams(use_tc_tiling_on_sc=True),
    out_type=jax.ShapeDtypeStruct((B,), jnp.float32),
    scratch_types=_SCRATCH,
)(_fmc_body)


def kernel(uid, basket_prev, iid, UI, IU, IL, LI):
    uid = uid.astype(jnp.int32)
    bp = basket_prev.astype(jnp.int32)
    iid = iid.astype(jnp.int32)
    mf_tab = _stage(UI.T, IU.T)
    fmc_tab = _stage(IL.T, LI.T)
    mf = _mf(uid, iid, mf_tab)
    return _fmc(iid, bp, fmc_tab, mf)


# speedup vs baseline: 2.9068x; 1.2603x over previous
import functools

import jax
import jax.numpy as jnp
from jax import lax
from jax.experimental import pallas as pl
from jax.experimental.pallas import tpu as pltpu
from jax.experimental.pallas import tpu_sc as plsc

K = 64
N = 1000000
B = 16384
NC = 2
NS = 16
NW = NC * NS
BPW = B // NW
CH = 128
L = 16
NCH = BPW // CH
NG = CH // L
SCALE = 1.0 / (K ** 0.5)

TBLK = 4096
TGRID = (N + TBLK - 1) // TBLK


def _stage_body(a_ref, b_ref, out_ref):
    out_ref[:, 0:K] = jnp.transpose(a_ref[...], (1, 0))
    out_ref[:, K:2 * K] = jnp.transpose(b_ref[...], (1, 0))


_stage = pl.pallas_call(
    _stage_body,
    grid=(TGRID,),
    in_specs=[pl.BlockSpec((K, TBLK), lambda i: (0, i)),
              pl.BlockSpec((K, TBLK), lambda i: (0, i))],
    out_specs=pl.BlockSpec((TBLK, 2 * K), lambda i: (i, 0)),
    out_shape=jax.ShapeDtypeStruct((N, 2 * K), jnp.float32),
)


def _pair_dots(idx_a_hbm, idx_b_hbm, tab, base, ia_v, ib_v, a_v, b_v,
               out_v, sem, combine):
    pltpu.sync_copy(idx_a_hbm.at[pl.ds(base, BPW)], ia_v)
    pltpu.sync_copy(idx_b_hbm.at[pl.ds(base, BPW)], ib_v)

    lanes = lax.iota(jnp.int32, L)

    def chunk_body(c, _):
        off = c * CH
        cp_a = pltpu.async_copy(tab.at[ia_v.at[pl.ds(off, CH)]], a_v, sem)
        cp_b = pltpu.async_copy(tab.at[ib_v.at[pl.ds(off, CH)]], b_v, sem)
        cp_a.wait()
        cp_b.wait()

        def grp_body(g, _):
            r0 = g * L
            vec = jnp.zeros((L,), jnp.float32)
            for r in range(L):
                acc = a_v[r0 + r, pl.ds(0, L)] * b_v[r0 + r, pl.ds(K, L)]
                for j in range(1, K // L):
                    acc = acc + (a_v[r0 + r, pl.ds(j * L, L)]
                                 * b_v[r0 + r, pl.ds(K + j * L, L)])
                for step in (8, 4, 2, 1):
                    acc = acc + acc.at[lanes ^ step].get(
                        mode="promise_in_bounds")
                vec = jnp.where(lanes == r, acc, vec)
            combine(pl.ds(off + r0, L), vec)
            return _

        return lax.fori_loop(0, NG, grp_body, None)

    lax.fori_loop(0, NCH, chunk_body, None)


def _mf_body(uid_hbm, iid_hbm, tab_hbm, out_hbm,
             ia_v, ib_v, a_v, b_v, out_v, sem):
    wid = lax.axis_index("s") * NC + lax.axis_index("c")
    base = wid * BPW

    def combine(dst, vec):
        out_v[dst] = vec

    _pair_dots(uid_hbm, iid_hbm, tab_hbm, base, ia_v, ib_v, a_v, b_v,
               out_v, sem, combine)
    pltpu.sync_copy(out_v, out_hbm.at[pl.ds(base, BPW)])


def _fmc_body(iid_hbm, bp_hbm, tab_hbm, mf_hbm, out_hbm,
              ia_v, ib_v, a_v, b_v, out_v, sem):
    wid = lax.axis_index("s") * NC + lax.axis_index("c")
    base = wid * BPW
    pltpu.sync_copy(mf_hbm.at[pl.ds(base, BPW)], out_v)

    def combine(dst, vec):
        out_v[dst] = (out_v[dst] + vec) * SCALE

    _pair_dots(iid_hbm, bp_hbm, tab_hbm, base, ia_v, ib_v, a_v, b_v,
               out_v, sem, combine)
    pltpu.sync_copy(out_v, out_hbm.at[pl.ds(base, BPW)])


_SCRATCH = [
    pltpu.VMEM((BPW,), jnp.int32),
    pltpu.VMEM((BPW,), jnp.int32),
    pltpu.VMEM((CH, 2 * K), jnp.float32),
    pltpu.VMEM((CH, 2 * K), jnp.float32),
    pltpu.VMEM((BPW,), jnp.float32),
    pltpu.SemaphoreType.DMA,
]

_mf = functools.partial(
    pl.kernel,
    mesh=plsc.VectorSubcoreMesh(core_axis_name="c", subcore_axis_name="s"),
    compiler_params=pltpu.CompilerParams(use_tc_tiling_on_sc=True),
    out_type=jax.ShapeDtypeStruct((B,), jnp.float32),
    scratch_types=_SCRATCH,
)(_mf_body)

_fmc = functools.partial(
    pl.kernel,
    mesh=plsc.VectorSubcoreMesh(core_axis_name="c", subcore_axis_name="s"),
    compiler_params=pltpu.CompilerParams(use_tc_tiling_on_sc=True),
    out_type=jax.ShapeDtypeStruct((B,), jnp.float32),
    scratch_types=_SCRATCH,
)(_fmc_body)


def kernel(uid, basket_prev, iid, UI, IU, IL, LI):
    uid = uid.astype(jnp.int32)
    bp = basket_prev.astype(jnp.int32)
    iid = iid.astype(jnp.int32)
    mf_tab = _stage(UI.T, IU.T)
    fmc_tab = _stage(IL.T, LI.T)
    mf = _mf(uid, iid, mf_tab)
    return _fmc(iid, bp, fmc_tab, mf)


# speedup vs baseline: 3.3471x; 1.1515x over previous
import functools

import jax
import jax.numpy as jnp
from jax import lax
from jax.experimental import pallas as pl
from jax.experimental.pallas import tpu as pltpu
from jax.experimental.pallas import tpu_sc as plsc

K = 64
N = 1000000
B = 16384
NC = 2
NS = 16
NW = NC * NS
BPW = B // NW
CH = 128
L = 16
NCH = BPW // CH
NG = CH // L
SCALE = 1.0 / (K ** 0.5)

TBLK = 8192
TGRID = (N + TBLK - 1) // TBLK


def _stage_body(a_ref, b_ref, out_ref):
    out_ref[:, 0:K] = jnp.transpose(a_ref[...], (1, 0))
    out_ref[:, K:2 * K] = jnp.transpose(b_ref[...], (1, 0))


_stage = pl.pallas_call(
    _stage_body,
    grid=(TGRID,),
    in_specs=[pl.BlockSpec((K, TBLK), lambda i: (0, i)),
              pl.BlockSpec((K, TBLK), lambda i: (0, i))],
    out_specs=pl.BlockSpec((TBLK, 2 * K), lambda i: (i, 0)),
    out_shape=jax.ShapeDtypeStruct((N, 2 * K), jnp.float32),
)


def _pair_dots(idx_a_hbm, idx_b_hbm, tab, base, ia_v, ib_v, a_v, b_v,
               out_v, sem, combine):
    pltpu.sync_copy(idx_a_hbm.at[pl.ds(base, BPW)], ia_v)
    pltpu.sync_copy(idx_b_hbm.at[pl.ds(base, BPW)], ib_v)

    lanes = lax.iota(jnp.int32, L)

    def chunk_body(c, _):
        off = c * CH
        cp_a = pltpu.async_copy(tab.at[ia_v.at[pl.ds(off, CH)]], a_v, sem)
        cp_b = pltpu.async_copy(tab.at[ib_v.at[pl.ds(off, CH)]], b_v, sem)
        cp_a.wait()
        cp_b.wait()

        def grp_body(g, _):
            r0 = g * L
            vec = jnp.zeros((L,), jnp.float32)
            for r in range(L):
                acc = a_v[r0 + r, pl.ds(0, L)] * b_v[r0 + r, pl.ds(K, L)]
                for j in range(1, K // L):
                    acc = acc + (a_v[r0 + r, pl.ds(j * L, L)]
                                 * b_v[r0 + r, pl.ds(K + j * L, L)])
                for step in (8, 4, 2, 1):
                    acc = acc + acc.at[lanes ^ step].get(
                        mode="promise_in_bounds")
                vec = jnp.where(lanes == r, acc, vec)
            combine(pl.ds(off + r0, L), vec)
            return _

        return lax.fori_loop(0, NG, grp_body, None)

    lax.fori_loop(0, NCH, chunk_body, None)


def _mf_body(uid_hbm, iid_hbm, tab_hbm, out_hbm,
             ia_v, ib_v, a_v, b_v, out_v, sem):
    wid = lax.axis_index("s") * NC + lax.axis_index("c")
    base = wid * BPW

    def combine(dst, vec):
        out_v[dst] = vec

    _pair_dots(uid_hbm, iid_hbm, tab_hbm, base, ia_v, ib_v, a_v, b_v,
               out_v, sem, combine)
    pltpu.sync_copy(out_v, out_hbm.at[pl.ds(base, BPW)])


def _fmc_body(iid_hbm, bp_hbm, tab_hbm, mf_hbm, out_hbm,
              ia_v, ib_v, a_v, b_v, out_v, sem):
    wid = lax.axis_index("s") * NC + lax.axis_index("c")
    base = wid * BPW
    pltpu.sync_copy(mf_hbm.at[pl.ds(base, BPW)], out_v)

    def combine(dst, vec):
        out_v[dst] = (out_v[dst] + vec) * SCALE

    _pair_dots(iid_hbm, bp_hbm, tab_hbm, base, ia_v, ib_v, a_v, b_v,
               out_v, sem, combine)
    pltpu.sync_copy(out_v, out_hbm.at[pl.ds(base, BPW)])


_SCRATCH = [
    pltpu.VMEM((BPW,), jnp.int32),
    pltpu.VMEM((BPW,), jnp.int32),
    pltpu.VMEM((CH, 2 * K), jnp.float32),
    pltpu.VMEM((CH, 2 * K), jnp.float32),
    pltpu.VMEM((BPW,), jnp.float32),
    pltpu.SemaphoreType.DMA,
]

_mf = functools.partial(
    pl.kernel,
    mesh=plsc.VectorSubcoreMesh(core_axis_name="c", subcore_axis_name="s"),
    compiler_params=pltpu.CompilerParams(use_tc_tiling_on_sc=True),
    out_type=jax.ShapeDtypeStruct((B,), jnp.float32),
    scratch_types=_SCRATCH,
)(_mf_body)

_fmc = functools.partial(
    pl.kernel,
    mesh=plsc.VectorSubcoreMesh(core_axis_name="c", subcore_axis_name="s"),
    compiler_params=pltpu.CompilerParams(use_tc_tiling_on_sc=True),
    out_type=jax.ShapeDtypeStruct((B,), jnp.float32),
    scratch_types=_SCRATCH,
)(_fmc_body)


def kernel(uid, basket_prev, iid, UI, IU, IL, LI):
    uid = uid.astype(jnp.int32)
    bp = basket_prev.astype(jnp.int32)
    iid = iid.astype(jnp.int32)
    mf_tab = _stage(UI.T, IU.T)
    fmc_tab = _stage(IL.T, LI.T)
    mf = _mf(uid, iid, mf_tab)
    return _fmc(iid, bp, fmc_tab, mf)


# speedup vs baseline: 3.5800x; 1.0696x over previous
import functools

import jax
import jax.numpy as jnp
from jax import lax
from jax.experimental import pallas as pl
from jax.experimental.pallas import tpu as pltpu
from jax.experimental.pallas import tpu_sc as plsc

K = 64
N = 1000000
B = 16384
NC = 2
NS = 16
NW = NC * NS
BPW = B // NW
CH = 128
L = 16
NCH = BPW // CH
NG = CH // L
SCALE = 1.0 / (K ** 0.5)

TBLK = 16384
TGRID = (N + TBLK - 1) // TBLK


def _stage_body(a_ref, b_ref, out_ref):
    out_ref[:, 0:K] = jnp.transpose(a_ref[...], (1, 0))
    out_ref[:, K:2 * K] = jnp.transpose(b_ref[...], (1, 0))


_stage = pl.pallas_call(
    _stage_body,
    grid=(TGRID,),
    in_specs=[pl.BlockSpec((K, TBLK), lambda i: (0, i)),
              pl.BlockSpec((K, TBLK), lambda i: (0, i))],
    out_specs=pl.BlockSpec((TBLK, 2 * K), lambda i: (i, 0)),
    out_shape=jax.ShapeDtypeStruct((N, 2 * K), jnp.float32),
)


def _pair_dots(idx_a_hbm, idx_b_hbm, tab, base, ia_v, ib_v, a_v, b_v,
               out_v, sem, combine):
    pltpu.sync_copy(idx_a_hbm.at[pl.ds(base, BPW)], ia_v)
    pltpu.sync_copy(idx_b_hbm.at[pl.ds(base, BPW)], ib_v)

    lanes = lax.iota(jnp.int32, L)

    def chunk_body(c, _):
        off = c * CH
        cp_a = pltpu.async_copy(tab.at[ia_v.at[pl.ds(off, CH)]], a_v, sem)
        cp_b = pltpu.async_copy(tab.at[ib_v.at[pl.ds(off, CH)]], b_v, sem)
        cp_a.wait()
        cp_b.wait()

        def grp_body(g, _):
            r0 = g * L
            vec = jnp.zeros((L,), jnp.float32)
            for r in range(L):
                acc = a_v[r0 + r, pl.ds(0, L)] * b_v[r0 + r, pl.ds(K, L)]
                for j in range(1, K // L):
                    acc = acc + (a_v[r0 + r, pl.ds(j * L, L)]
                                 * b_v[r0 + r, pl.ds(K + j * L, L)])
                for step in (8, 4, 2, 1):
                    acc = acc + acc.at[lanes ^ step].get(
                        mode="promise_in_bounds")
                vec = jnp.where(lanes == r, acc, vec)
            combine(pl.ds(off + r0, L), vec)
            return _

        return lax.fori_loop(0, NG, grp_body, None)

    lax.fori_loop(0, NCH, chunk_body, None)


def _mf_body(uid_hbm, iid_hbm, tab_hbm, out_hbm,
             ia_v, ib_v, a_v, b_v, out_v, sem):
    wid = lax.axis_index("s") * NC + lax.axis_index("c")
    base = wid * BPW

    def combine(dst, vec):
        out_v[dst] = vec

    _pair_dots(uid_hbm, iid_hbm, tab_hbm, base, ia_v, ib_v, a_v, b_v,
               out_v, sem, combine)
    pltpu.sync_copy(out_v, out_hbm.at[pl.ds(base, BPW)])


def _fmc_body(iid_hbm, bp_hbm, tab_hbm, mf_hbm, out_hbm,
              ia_v, ib_v, a_v, b_v, out_v, sem):
    wid = lax.axis_index("s") * NC + lax.axis_index("c")
    base = wid * BPW
    pltpu.sync_copy(mf_hbm.at[pl.ds(base, BPW)], out_v)

    def combine(dst, vec):
        out_v[dst] = (out_v[dst] + vec) * SCALE

    _pair_dots(iid_hbm, bp_hbm, tab_hbm, base, ia_v, ib_v, a_v, b_v,
               out_v, sem, combine)
    pltpu.sync_copy(out_v, out_hbm.at[pl.ds(base, BPW)])


_SCRATCH = [
    pltpu.VMEM((BPW,), jnp.int32),
    pltpu.VMEM((BPW,), jnp.int32),
    pltpu.VMEM((CH, 2 * K), jnp.float32),
    pltpu.VMEM((CH, 2 * K), jnp.float32),
    pltpu.VMEM((BPW,), jnp.float32),
    pltpu.SemaphoreType.DMA,
]

_mf = functools.partial(
    pl.kernel,
    mesh=plsc.VectorSubcoreMesh(core_axis_name="c", subcore_axis_name="s"),
    compiler_params=pltpu.CompilerParams(use_tc_tiling_on_sc=True),
    out_type=jax.ShapeDtypeStruct((B,), jnp.float32),
    scratch_types=_SCRATCH,
)(_mf_body)

_fmc = functools.partial(
    pl.kernel,
    mesh=plsc.VectorSubcoreMesh(core_axis_name="c", subcore_axis_name="s"),
    compiler_params=pltpu.CompilerParams(use_tc_tiling_on_sc=True),
    out_type=jax.ShapeDtypeStruct((B,), jnp.float32),
    scratch_types=_SCRATCH,
)(_fmc_body)


def kernel(uid, basket_prev, iid, UI, IU, IL, LI):
    uid = uid.astype(jnp.int32)
    bp = basket_prev.astype(jnp.int32)
    iid = iid.astype(jnp.int32)
    mf_tab = _stage(UI.T, IU.T)
    fmc_tab = _stage(IL.T, LI.T)
    mf = _mf(uid, iid, mf_tab)
    return _fmc(iid, bp, fmc_tab, mf)
